# baseline (speedup 1.0000x reference)
import functools

import jax
import jax.numpy as jnp
from jax import lax
from jax.experimental import pallas as pl
from jax.experimental.pallas import tpu as pltpu
from jax.experimental.pallas import tpu_sc as plsc

RADIUS2 = 0.2 * 0.2
NSAMPLE = 32

B, N, NPOINT, C = 4, 16384, 1024, 64
NCHUNK = N // 16

try:
    _info = plsc.get_sparse_core_info()
    NC, NS = _info.num_cores, _info.num_subcores
except Exception:
    NC, NS = 2, 16
NW = NC * NS
TPB = NW // B
CPT = NPOINT // TPB
FPT = C // TPB

_mesh = plsc.VectorSubcoreMesh(core_axis_name="c", subcore_axis_name="s",
                               num_cores=NC, num_subcores=NS)


def _rnd_bf16(x):
    u = plsc.bitcast(x, jnp.uint32)
    r = u + jnp.uint32(0x7FFF) + ((u >> 16) & jnp.uint32(1))
    return plsc.bitcast(r & jnp.uint32(0xFFFF0000), jnp.float32)


@functools.partial(
    pl.kernel,
    out_type=jax.ShapeDtypeStruct((B, NPOINT, NSAMPLE), jnp.int32),
    mesh=_mesh,
    compiler_params=pltpu.CompilerParams(needs_layout_passes=False),
    scratch_types=[
        pltpu.VMEM((3 * 2048,), jnp.float32),
        pltpu.VMEM((N,), jnp.float32),
        pltpu.VMEM((N,), jnp.float32),
        pltpu.VMEM((N,), jnp.float32),
        pltpu.VMEM((N,), jnp.float32),
        pltpu.VMEM((3 * CPT,), jnp.float32),
        pltpu.VMEM((128,), jnp.int32),
        pltpu.VMEM((CPT, NSAMPLE), jnp.int32),
    ],
)
def _ball_query_sc(xyz_hbm, nxyz_hbm, idx_hbm, raw3, xv, yv, zv, ppv, rawc,
                   hits, idxout):
    wid = lax.axis_index("s") * NC + lax.axis_index("c")
    b = wid // TPB
    seg = wid % TPB
    pltpu.sync_copy(nxyz_hbm.at[b, pl.ds(seg * 3 * CPT, 3 * CPT)], rawc)

    lanes = lax.iota(jnp.int32, 16)
    lanes3 = lanes * 3
    r2 = jnp.full((16,), RADIUS2, jnp.float32)

    SEG = 2048
    for s in range(N // SEG):
        pltpu.sync_copy(xyz_hbm.at[b, pl.ds(s * 3 * SEG, 3 * SEG)], raw3)

        def dein(j):
            bb = j * 16
            iv = lanes3 + (3 * bb)
            xo = plsc.load_gather(raw3, [iv])
            yo = plsc.load_gather(raw3, [iv + 1])
            zo = plsc.load_gather(raw3, [iv + 2])
            gb = s * SEG + bb
            ppv[pl.ds(gb, 16)] = (xo * xo + yo * yo) + zo * zo
            xv[pl.ds(gb, 16)] = _rnd_bf16(xo)
            yv[pl.ds(gb, 16)] = _rnd_bf16(yo)
            zv[pl.ds(gb, 16)] = _rnd_bf16(zo)

        plsc.parallel_loop(0, SEG // 16, 1, unroll=4)(dein)

    def per_centroid(p, _):
        p3 = 3 * p
        cxo = plsc.load_gather(rawc, [jnp.full((16,), p3, jnp.int32)])
        cyo = plsc.load_gather(rawc, [jnp.full((16,), p3 + 1, jnp.int32)])
        czo = plsc.load_gather(rawc, [jnp.full((16,), p3 + 2, jnp.int32)])
        cc = (cxo * cxo + cyo * cyo) + czo * czo
        cx = _rnd_bf16(cxo)
        cy = _rnd_bf16(cyo)
        cz = _rnd_bf16(czo)

        def cond(carry):
            i, cnt = carry
            return (cnt < NSAMPLE) & (i < NCHUNK // 4)

        def chunk(base, cnt):
            px = xv[pl.ds(base, 16)]
            py = yv[pl.ds(base, 16)]
            pz = zv[pl.ds(base, 16)]
            pp = ppv[pl.ds(base, 16)]
            dot = (cx * px + cy * py) + cz * pz
            d2 = (cc + pp) - (dot + dot)
            m = d2 <= r2
            plsc.store_compressed(hits.at[pl.ds(cnt, 16)], base + lanes,
                                  mask=m)
            return cnt + plsc.all_reduce_population_count(m)[0]

        def body(carry):
            i, cnt = carry
            base = i * 64
            cnt = chunk(base, cnt)
            cnt = chunk(base + 16, cnt)
            cnt = chunk(base + 32, cnt)
            cnt = chunk(base + 48, cnt)
            return i + 1, cnt

        _, cnt = lax.while_loop(cond, body, (jnp.int32(0), jnp.int32(0)))

        pad = plsc.load_gather(hits, [jnp.zeros((16,), jnp.int32)])
        h0 = hits[pl.ds(0, 16)]
        h1 = hits[pl.ds(16, 16)]
        idxout[p, pl.ds(0, 16)] = jnp.where(lanes < cnt, h0, pad)
        idxout[p, pl.ds(16, 16)] = jnp.where(lanes + 16 < cnt, h1, pad)
        return ()

    lax.fori_loop(0, CPT, per_centroid, ())
    pltpu.sync_copy(idxout, idx_hbm.at[b, pl.ds(seg * CPT, CPT)])


@functools.partial(
    pl.kernel,
    out_type=jax.ShapeDtypeStruct((B, 6 + C, NPOINT * NSAMPLE), jnp.float32),
    mesh=_mesh,
    compiler_params=pltpu.CompilerParams(needs_layout_passes=False),
    scratch_types=[
        pltpu.VMEM((NPOINT * NSAMPLE // 2,), jnp.int32),
        pltpu.VMEM((3 * N,), jnp.float32),
        pltpu.VMEM((3 * NPOINT,), jnp.float32),
        pltpu.VMEM((NPOINT * NSAMPLE // 2,), jnp.float32),
    ],
)
def _group_sc(idx_hbm, xyz_hbm, nxyz_hbm, feat_hbm, out_hbm,
              idxv, buf, rawc, outbuf):
    wid = lax.axis_index("s") * NC + lax.axis_index("c")
    b = wid // TPB
    r = wid % TPB
    HALF = NPOINT * NSAMPLE // 2

    def gather_row():
        def gstep(j):
            iv = idxv[pl.ds(j * 16, 16)]
            outbuf[pl.ds(j * 16, 16)] = plsc.load_gather(buf, [iv])
        plsc.parallel_loop(0, HALF // 16, 1, unroll=8)(gstep)

    for h in range(2):
        pltpu.sync_copy(idx_hbm.at[b, pl.ds(h * HALF, HALF)], idxv)

        for k in range(FPT):
            ch = r + k * TPB
            pltpu.sync_copy(feat_hbm.at[b, ch], buf.at[pl.ds(0, N)])
            gather_row()
            pltpu.sync_copy(outbuf, out_hbm.at[b, 6 + ch, pl.ds(h * HALF, HALF)])

        @pl.when(r < 3)
        def _():
            d = r
            pltpu.sync_copy(xyz_hbm.at[b], buf)
            pltpu.sync_copy(nxyz_hbm.at[b], rawc)
            pbase = h * (NPOINT // 2)

            def cstep(p):
                pv = jnp.full((16,), 3 * (pbase + p) + d, jnp.int32)
                cb = plsc.load_gather(rawc, [pv])
                iv0 = idxv[pl.ds(p * NSAMPLE, 16)] * 3 + d
                iv1 = idxv[pl.ds(p * NSAMPLE + 16, 16)] * 3 + d
                outbuf[pl.ds(p * NSAMPLE, 16)] = (
                    plsc.load_gather(buf, [iv0]) - cb)
                outbuf[pl.ds(p * NSAMPLE + 16, 16)] = (
                    plsc.load_gather(buf, [iv1]) - cb)

            plsc.parallel_loop(0, NPOINT // 2, 1, unroll=4)(cstep)
            pltpu.sync_copy(outbuf, out_hbm.at[b, d, pl.ds(h * HALF, HALF)])
            pltpu.sync_copy(outbuf, out_hbm.at[b, 3 + d, pl.ds(h * HALF, HALF)])


def kernel(xyz, new_xyz, features):
    xyzf = xyz.reshape(B, 3 * N)
    nxyzf = new_xyz.reshape(B, 3 * NPOINT)
    idx = _ball_query_sc(xyzf, nxyzf)
    idx2 = idx.reshape(B, NPOINT * NSAMPLE)
    out = _group_sc(idx2, xyzf, nxyzf, features)
    return out.reshape(B, 6 + C, NPOINT, NSAMPLE)


# speedup vs baseline: 1.6594x; 1.6594x over previous
import functools

import jax
import jax.numpy as jnp
from jax import lax
from jax.experimental import pallas as pl
from jax.experimental.pallas import tpu as pltpu
from jax.experimental.pallas import tpu_sc as plsc

RADIUS2 = 0.2 * 0.2
NSAMPLE = 32

B, N, NPOINT, C = 4, 16384, 1024, 64
NCHUNK = N // 16

try:
    _info = plsc.get_sparse_core_info()
    NC, NS = _info.num_cores, _info.num_subcores
except Exception:
    NC, NS = 2, 16
NW = NC * NS
TPB = NW // B
CPT = NPOINT // TPB
FPT = C // TPB

_mesh = plsc.VectorSubcoreMesh(core_axis_name="c", subcore_axis_name="s",
                               num_cores=NC, num_subcores=NS)


@functools.partial(
    pl.kernel,
    out_type=jax.ShapeDtypeStruct((B, NSAMPLE, NPOINT), jnp.int32),
    mesh=_mesh,
    compiler_params=pltpu.CompilerParams(needs_layout_passes=False),
    scratch_types=[
        pltpu.VMEM((N,), jnp.float32),
        pltpu.VMEM((N,), jnp.float32),
        pltpu.VMEM((N,), jnp.float32),
        pltpu.VMEM((N,), jnp.float32),
        pltpu.VMEM((CPT,), jnp.float32),
        pltpu.VMEM((CPT,), jnp.float32),
        pltpu.VMEM((CPT,), jnp.float32),
        pltpu.VMEM((CPT,), jnp.float32),
        pltpu.VMEM((128,), jnp.int32),
        pltpu.VMEM((NSAMPLE, CPT), jnp.int32),
    ],
)
def _ball_query_sc(xb_hbm, yb_hbm, zb_hbm, pp_hbm, cx_hbm, cy_hbm, cz_hbm,
                   cc_hbm, idx_hbm, xv, yv, zv, ppv, cxv, cyv, czv, ccv,
                   hits, idxout):
    wid = lax.axis_index("s") * NC + lax.axis_index("c")
    b = wid // TPB
    seg = wid % TPB
    pltpu.sync_copy(xb_hbm.at[b], xv)
    pltpu.sync_copy(yb_hbm.at[b], yv)
    pltpu.sync_copy(zb_hbm.at[b], zv)
    pltpu.sync_copy(pp_hbm.at[b], ppv)
    pltpu.sync_copy(cx_hbm.at[b, pl.ds(seg * CPT, CPT)], cxv)
    pltpu.sync_copy(cy_hbm.at[b, pl.ds(seg * CPT, CPT)], cyv)
    pltpu.sync_copy(cz_hbm.at[b, pl.ds(seg * CPT, CPT)], czv)
    pltpu.sync_copy(cc_hbm.at[b, pl.ds(seg * CPT, CPT)], ccv)

    lanes = lax.iota(jnp.int32, 16)
    r2 = jnp.full((16,), RADIUS2, jnp.float32)

    def per_centroid(p, _):
        pv = jnp.full((16,), p, jnp.int32)
        cx = plsc.load_gather(cxv, [pv])
        cy = plsc.load_gather(cyv, [pv])
        cz = plsc.load_gather(czv, [pv])
        cc = plsc.load_gather(ccv, [pv])

        def cond(carry):
            i, cnt = carry
            return (cnt < NSAMPLE) & (i < NCHUNK // 4)

        def chunk(base, cnt):
            px = xv[pl.ds(base, 16)]
            py = yv[pl.ds(base, 16)]
            pz = zv[pl.ds(base, 16)]
            pp = ppv[pl.ds(base, 16)]
            dot = (cx * px + cy * py) + cz * pz
            d2 = (cc + pp) - (dot + dot)
            m = d2 <= r2
            plsc.store_compressed(hits.at[pl.ds(cnt, 16)], base + lanes,
                                  mask=m)
            return cnt + plsc.all_reduce_population_count(m)[0]

        def body(carry):
            i, cnt = carry
            base = i * 64
            cnt = chunk(base, cnt)
            cnt = chunk(base + 16, cnt)
            cnt = chunk(base + 32, cnt)
            cnt = chunk(base + 48, cnt)
            return i + 1, cnt

        _, cnt = lax.while_loop(cond, body, (jnp.int32(0), jnp.int32(0)))

        pad = plsc.load_gather(hits, [jnp.zeros((16,), jnp.int32)])
        h0 = hits[pl.ds(0, 16)]
        h1 = hits[pl.ds(16, 16)]
        plsc.store_scatter(idxout, [lanes, pv],
                           jnp.where(lanes < cnt, h0, pad))
        plsc.store_scatter(idxout, [lanes + 16, pv],
                           jnp.where(lanes + 16 < cnt, h1, pad))
        return ()

    lax.fori_loop(0, CPT, per_centroid, ())
    pltpu.sync_copy(idxout, idx_hbm.at[b, :, pl.ds(seg * CPT, CPT)])


@functools.partial(
    pl.kernel,
    out_type=jax.ShapeDtypeStruct((B, 6 + C, NSAMPLE, NPOINT), jnp.float32),
    mesh=_mesh,
    compiler_params=pltpu.CompilerParams(needs_layout_passes=False),
    scratch_types=[
        pltpu.VMEM((NSAMPLE, NPOINT), jnp.int32),
        pltpu.VMEM((N,), jnp.float32),
        pltpu.VMEM((NPOINT,), jnp.float32),
        pltpu.VMEM((NSAMPLE, NPOINT), jnp.float32),
    ],
)
def _group_sc(idx_hbm, xyzt_hbm, cent_hbm, feat_hbm, out_hbm,
              idxv, row, cv, outbuf):
    wid = lax.axis_index("s") * NC + lax.axis_index("c")
    b = wid // TPB
    r = wid % TPB
    pltpu.sync_copy(idx_hbm.at[b], idxv)
    lanes = lax.iota(jnp.int32, 16)
    NCOL = NPOINT // 16

    def gather_row():
        def gstep(j):
            s = j // NCOL
            c = j % NCOL
            iv = idxv[s, pl.ds(c * 16, 16)]
            outbuf[s, pl.ds(c * 16, 16)] = plsc.load_gather(row, [iv])
        plsc.parallel_loop(0, NSAMPLE * NCOL, 1, unroll=8)(gstep)

    for k in range(FPT):
        ch = r + k * TPB
        pltpu.sync_copy(feat_hbm.at[b, ch], row)
        gather_row()
        pltpu.sync_copy(outbuf, out_hbm.at[b, 6 + ch])

    @pl.when(r < 3)
    def _():
        d = r
        pltpu.sync_copy(xyzt_hbm.at[b, d], row)
        pltpu.sync_copy(cent_hbm.at[b, d], cv)

        def cstep(j):
            s = j // NCOL
            c = j % NCOL
            iv = idxv[s, pl.ds(c * 16, 16)]
            cb = plsc.load_gather(cv, [c * 16 + lanes])
            outbuf[s, pl.ds(c * 16, 16)] = plsc.load_gather(row, [iv]) - cb

        plsc.parallel_loop(0, NSAMPLE * NCOL, 1, unroll=8)(cstep)
        pltpu.sync_copy(outbuf, out_hbm.at[b, d])
        pltpu.sync_copy(outbuf, out_hbm.at[b, 3 + d])


def _round_bf16(x):
    u = lax.bitcast_convert_type(x, jnp.uint32)
    r = u + jnp.uint32(0x7FFF) + ((u >> 16) & jnp.uint32(1))
    return lax.bitcast_convert_type(r & jnp.uint32(0xFFFF0000), jnp.float32)


def kernel(xyz, new_xyz, features):
    xb = _round_bf16(xyz)
    nb = _round_bf16(new_xyz)
    cc = jnp.sum(new_xyz * new_xyz, axis=-1)
    pp = jnp.sum(xyz * xyz, axis=-1)
    xbx, xby, xbz = xb[..., 0], xb[..., 1], xb[..., 2]
    nbx, nby, nbz = nb[..., 0], nb[..., 1], nb[..., 2]
    idx = _ball_query_sc(xbx, xby, xbz, pp, nbx, nby, nbz, cc)
    xyzt = jnp.transpose(xyz, (0, 2, 1))
    cent = jnp.transpose(new_xyz, (0, 2, 1))
    out = _group_sc(idx, xyzt, cent, features)
    return jnp.transpose(out, (0, 1, 3, 2))


# speedup vs baseline: 2.2335x; 1.3460x over previous
import functools

import jax
import jax.numpy as jnp
from jax import lax
from jax.experimental import pallas as pl
from jax.experimental.pallas import tpu as pltpu
from jax.experimental.pallas import tpu_sc as plsc

RADIUS2 = 0.2 * 0.2
NSAMPLE = 32

B, N, NPOINT, C = 4, 16384, 1024, 64
NCHUNK = N // 16

try:
    _info = plsc.get_sparse_core_info()
    NC, NS = _info.num_cores, _info.num_subcores
except Exception:
    NC, NS = 2, 16
NW = NC * NS
TPB = NW // B
CPT = NPOINT // TPB
FPT = C // TPB

_mesh = plsc.VectorSubcoreMesh(core_axis_name="c", subcore_axis_name="s",
                               num_cores=NC, num_subcores=NS)


@functools.partial(
    pl.kernel,
    out_type=jax.ShapeDtypeStruct((B, NSAMPLE, NPOINT), jnp.int32),
    mesh=_mesh,
    compiler_params=pltpu.CompilerParams(needs_layout_passes=False),
    scratch_types=[
        pltpu.VMEM((N,), jnp.float32),
        pltpu.VMEM((N,), jnp.float32),
        pltpu.VMEM((N,), jnp.float32),
        pltpu.VMEM((N,), jnp.float32),
        pltpu.VMEM((CPT,), jnp.float32),
        pltpu.VMEM((CPT,), jnp.float32),
        pltpu.VMEM((CPT,), jnp.float32),
        pltpu.VMEM((CPT,), jnp.float32),
        pltpu.VMEM((128,), jnp.int32),
        pltpu.VMEM((128,), jnp.int32),
        pltpu.VMEM((128,), jnp.int32),
        pltpu.VMEM((128,), jnp.int32),
        pltpu.VMEM((NSAMPLE, CPT), jnp.int32),
    ],
)
def _ball_query_sc(xb_hbm, yb_hbm, zb_hbm, pp_hbm, cx_hbm, cy_hbm, cz_hbm,
                   cc_hbm, idx_hbm, xv, yv, zv, ppv, cxv, cyv, czv, ccv,
                   hits0, hits1, hits2, hits3, idxout):
    wid = lax.axis_index("s") * NC + lax.axis_index("c")
    b = wid // TPB
    seg = wid % TPB
    pltpu.sync_copy(xb_hbm.at[b], xv)
    pltpu.sync_copy(yb_hbm.at[b], yv)
    pltpu.sync_copy(zb_hbm.at[b], zv)
    pltpu.sync_copy(pp_hbm.at[b], ppv)
    pltpu.sync_copy(cx_hbm.at[b, pl.ds(seg * CPT, CPT)], cxv)
    pltpu.sync_copy(cy_hbm.at[b, pl.ds(seg * CPT, CPT)], cyv)
    pltpu.sync_copy(cz_hbm.at[b, pl.ds(seg * CPT, CPT)], czv)
    pltpu.sync_copy(cc_hbm.at[b, pl.ds(seg * CPT, CPT)], ccv)

    lanes = lax.iota(jnp.int32, 16)
    r2 = jnp.full((16,), RADIUS2, jnp.float32)
    dead = jnp.full((16,), -1e30, jnp.float32)
    G = 4
    hits = (hits0, hits1, hits2, hits3)

    def per_group(g, _):
        cxs, cys, czs, ccs = [], [], [], []
        for q in range(G):
            pv = jnp.full((16,), g * G + q, jnp.int32)
            cxs.append(plsc.load_gather(cxv, [pv]))
            cys.append(plsc.load_gather(cyv, [pv]))
            czs.append(plsc.load_gather(czv, [pv]))
            ccs.append(plsc.load_gather(ccv, [pv]))

        def cond(carry):
            i = carry[0]
            cs = carry[1:]
            alive = cs[0] < NSAMPLE
            for q in range(1, G):
                alive = alive | (cs[q] < NSAMPLE)
            return alive & (i < NCHUNK // 4)

        def body(carry):
            i = carry[0]
            cs = list(carry[1:])
            base0 = i * 64
            r2e = [jnp.where(cs[q] < NSAMPLE, r2, dead) for q in range(G)]
            for c in range(4):
                base = base0 + c * 16
                px = xv[pl.ds(base, 16)]
                py = yv[pl.ds(base, 16)]
                pz = zv[pl.ds(base, 16)]
                pp = ppv[pl.ds(base, 16)]
                for q in range(G):
                    dot = (cxs[q] * px + cys[q] * py) + czs[q] * pz
                    d2 = (ccs[q] + pp) - (dot + dot)
                    m = d2 <= r2e[q]
                    plsc.store_compressed(hits[q].at[pl.ds(cs[q], 16)],
                                          base + lanes, mask=m)
                    cs[q] = cs[q] + plsc.all_reduce_population_count(m)[0]
            return (i + 1, *cs)

        carry = lax.while_loop(cond, body,
                               (jnp.int32(0),) + (jnp.int32(0),) * G)
        cs = carry[1:]

        for q in range(G):
            pv = jnp.full((16,), g * G + q, jnp.int32)
            pad = plsc.load_gather(hits[q], [jnp.zeros((16,), jnp.int32)])
            h0 = hits[q][pl.ds(0, 16)]
            h1 = hits[q][pl.ds(16, 16)]
            plsc.store_scatter(idxout, [lanes, pv],
                               jnp.where(lanes < cs[q], h0, pad))
            plsc.store_scatter(idxout, [lanes + 16, pv],
                               jnp.where(lanes + 16 < cs[q], h1, pad))
        return ()

    lax.fori_loop(0, CPT // G, per_group, ())
    pltpu.sync_copy(idxout, idx_hbm.at[b, :, pl.ds(seg * CPT, CPT)])


@functools.partial(
    pl.kernel,
    out_type=jax.ShapeDtypeStruct((B, 6 + C, NSAMPLE, NPOINT), jnp.float32),
    mesh=_mesh,
    compiler_params=pltpu.CompilerParams(needs_layout_passes=False),
    scratch_types=[
        pltpu.VMEM((NSAMPLE, NPOINT), jnp.int32),
        pltpu.VMEM((N,), jnp.float32),
        pltpu.VMEM((NPOINT,), jnp.float32),
        pltpu.VMEM((NSAMPLE, NPOINT), jnp.float32),
    ],
)
def _group_sc(idx_hbm, xyzt_hbm, cent_hbm, feat_hbm, out_hbm,
              idxv, row, cv, outbuf):
    wid = lax.axis_index("s") * NC + lax.axis_index("c")
    b = wid // TPB
    r = wid % TPB
    pltpu.sync_copy(idx_hbm.at[b], idxv)
    lanes = lax.iota(jnp.int32, 16)
    NCOL = NPOINT // 16

    def gather_row():
        def gstep(j):
            s = j // NCOL
            c = j % NCOL
            iv = idxv[s, pl.ds(c * 16, 16)]
            outbuf[s, pl.ds(c * 16, 16)] = plsc.load_gather(row, [iv])
        plsc.parallel_loop(0, NSAMPLE * NCOL, 1, unroll=8)(gstep)

    for k in range(FPT):
        ch = r + k * TPB
        pltpu.sync_copy(feat_hbm.at[b, ch], row)
        gather_row()
        pltpu.sync_copy(outbuf, out_hbm.at[b, 6 + ch])

    @pl.when(r < 3)
    def _():
        d = r
        pltpu.sync_copy(xyzt_hbm.at[b, d], row)
        pltpu.sync_copy(cent_hbm.at[b, d], cv)

        def cstep(j):
            s = j // NCOL
            c = j % NCOL
            iv = idxv[s, pl.ds(c * 16, 16)]
            cb = plsc.load_gather(cv, [c * 16 + lanes])
            outbuf[s, pl.ds(c * 16, 16)] = plsc.load_gather(row, [iv]) - cb

        plsc.parallel_loop(0, NSAMPLE * NCOL, 1, unroll=8)(cstep)
        pltpu.sync_copy(outbuf, out_hbm.at[b, d])
        pltpu.sync_copy(outbuf, out_hbm.at[b, 3 + d])


def _round_bf16(x):
    u = lax.bitcast_convert_type(x, jnp.uint32)
    r = u + jnp.uint32(0x7FFF) + ((u >> 16) & jnp.uint32(1))
    return lax.bitcast_convert_type(r & jnp.uint32(0xFFFF0000), jnp.float32)


def kernel(xyz, new_xyz, features):
    xb = _round_bf16(xyz)
    nb = _round_bf16(new_xyz)
    cc = jnp.sum(new_xyz * new_xyz, axis=-1)
    pp = jnp.sum(xyz * xyz, axis=-1)
    xbx, xby, xbz = xb[..., 0], xb[..., 1], xb[..., 2]
    nbx, nby, nbz = nb[..., 0], nb[..., 1], nb[..., 2]
    idx = _ball_query_sc(xbx, xby, xbz, pp, nbx, nby, nbz, cc)
    xyzt = jnp.transpose(xyz, (0, 2, 1))
    cent = jnp.transpose(new_xyz, (0, 2, 1))
    out = _group_sc(idx, xyzt, cent, features)
    return jnp.transpose(out, (0, 1, 3, 2))
